# Initial kernel scaffold; baseline (speedup 1.0000x reference)
import jax
import jax.numpy as jnp
from jax.experimental import pallas as pl


def kernel(x, edge_idx, W1, b1, W2, b2, W3, b3):
    raise NotImplementedError("write your pallas kernel here")


# speedup vs baseline: 2.2476x; 2.2476x over previous
import functools

import jax
import jax.numpy as jnp
from jax import lax
from jax.experimental import pallas as pl
from jax.experimental.pallas import tpu as pltpu
from jax.experimental.pallas import tpu_sc as plsc

N_NODES = 10000
N_EDGES = 320000
D = 128

NC = 2
NS = 16
NW = NC * NS
EPW = N_EDGES // NW
C = 80
NP = 10112
N_PER_TILE = NP // NS


def _preproj_body(x_ref, w1_ref, b1_ref, out_ref):
    xa = jnp.dot(x_ref[...], w1_ref[:D, :], preferred_element_type=jnp.float32)
    xb = jnp.dot(x_ref[...], w1_ref[D:, :], preferred_element_type=jnp.float32)
    out_ref[0] = xa
    out_ref[1] = xb + b1_ref[...]


def _preproj(x, W1, b1):
    BN = 2000
    grid = (N_NODES // BN,)
    return pl.pallas_call(
        _preproj_body,
        grid=grid,
        in_specs=[
            pl.BlockSpec((BN, D), lambda i: (i, 0)),
            pl.BlockSpec((2 * D, D), lambda i: (0, 0)),
            pl.BlockSpec((1, D), lambda i: (0, 0)),
        ],
        out_specs=pl.BlockSpec((2, BN, D), lambda i: (0, i, 0)),
        out_shape=jax.ShapeDtypeStruct((2, N_NODES, D), jnp.float32),
    )(x, W1, b1.reshape(1, D))


def _gather_body(tab_hbm, ridx_hbm, cidx_hbm, out_hbm,
                 ia_v, ib_v, ba_v, bb_v, sa, sb):
    wid = lax.axis_index("s") * NC + lax.axis_index("c")
    e0 = wid * EPW

    def chunk(ci, carry):
        base = e0 + ci * C
        pltpu.sync_copy(ridx_hbm.at[pl.ds(base, C)], ia_v)
        pltpu.sync_copy(cidx_hbm.at[pl.ds(base, C)], ib_v)
        cpa = pltpu.async_copy(tab_hbm.at[ia_v], ba_v, sa)
        cpb = pltpu.async_copy(tab_hbm.at[ib_v], bb_v, sb)
        cpa.wait()
        cpb.wait()

        def row_add(i, c2):
            for j in range(D // 16):
                sl = pl.ds(j * 16, 16)
                ba_v[i, sl] = ba_v[i, sl] + bb_v[i, sl]
            return c2

        lax.fori_loop(0, C, row_add, 0, unroll=2)
        pltpu.sync_copy(ba_v, out_hbm.at[pl.ds(base, C)])
        return carry

    lax.fori_loop(0, EPW // C, chunk, 0)


def _gather_add(xab, ridx, cidxp):
    mesh = plsc.VectorSubcoreMesh(core_axis_name="c", subcore_axis_name="s")
    f = pl.kernel(
        _gather_body,
        out_type=jax.ShapeDtypeStruct((N_EDGES, D), jnp.float32),
        mesh=mesh,
        scratch_types=[
            pltpu.VMEM((C,), jnp.int32),
            pltpu.VMEM((C,), jnp.int32),
            pltpu.VMEM((C, D), jnp.float32),
            pltpu.VMEM((C, D), jnp.float32),
            pltpu.SemaphoreType.DMA,
            pltpu.SemaphoreType.DMA,
        ],
    )
    return f(xab, ridx, cidxp)


def _softplus(h):
    return jnp.maximum(h, 0.0) + jnp.log(1.0 + jnp.exp(-jnp.abs(h)))


def _mlp_body(h_ref, w2_ref, b2_ref, w3_ref, b3_ref, out_ref):
    h = _softplus(h_ref[...])
    h = _softplus(jnp.dot(h, w2_ref[...], preferred_element_type=jnp.float32)
                  + b2_ref[...])
    out_ref[...] = (jnp.dot(h, w3_ref[...], preferred_element_type=jnp.float32)
                    + b3_ref[...])


def _mlp_tail(hpre, W2, b2, W3, b3):
    BE = 3200
    grid = (N_EDGES // BE,)
    return pl.pallas_call(
        _mlp_body,
        grid=grid,
        in_specs=[
            pl.BlockSpec((BE, D), lambda i: (i, 0)),
            pl.BlockSpec((D, D), lambda i: (0, 0)),
            pl.BlockSpec((1, D), lambda i: (0, 0)),
            pl.BlockSpec((D, D), lambda i: (0, 0)),
            pl.BlockSpec((1, D), lambda i: (0, 0)),
        ],
        out_specs=pl.BlockSpec((BE, D), lambda i: (i, 0)),
        out_shape=jax.ShapeDtypeStruct((N_EDGES, D), jnp.float32),
    )(hpre, W2, b2.reshape(1, D), W3, b3.reshape(1, D))


def _scatter_body(h3_hbm, ridx_hbm, out_hbm, idx_v, buf_v, zbuf_v, accum_sh, sem):
    cid = lax.axis_index("c")
    sid = lax.axis_index("s")
    wid = sid * NC + cid

    zeros16 = jnp.zeros((16,), jnp.float32)
    for i in range(8):
        for j in range(D // 16):
            zbuf_v[i, pl.ds(j * 16, 16)] = zeros16
    r0 = sid * N_PER_TILE

    def zchunk(k, c2):
        pltpu.sync_copy(zbuf_v, accum_sh.at[pl.ds(r0 + k * 8, 8)])
        return c2

    lax.fori_loop(0, N_PER_TILE // 8, zchunk, 0)
    plsc.subcore_barrier()

    e0 = wid * EPW

    def chunk(ci, carry):
        base = e0 + ci * C
        pltpu.sync_copy(ridx_hbm.at[pl.ds(base, C)], idx_v)
        pltpu.sync_copy(h3_hbm.at[pl.ds(base, C)], buf_v)
        pltpu.sync_copy(buf_v, accum_sh.at[idx_v], add=True)
        return carry

    lax.fori_loop(0, EPW // C, chunk, 0)
    plsc.subcore_barrier()

    pltpu.sync_copy(accum_sh.at[pl.ds(r0, N_PER_TILE)],
                    out_hbm.at[cid, pl.ds(r0, N_PER_TILE)])


def _segment_sum(h3, ridx):
    mesh = plsc.VectorSubcoreMesh(core_axis_name="c", subcore_axis_name="s")
    f = pl.kernel(
        _scatter_body,
        out_type=jax.ShapeDtypeStruct((NC, NP, D), jnp.float32),
        mesh=mesh,
        scratch_types=[
            pltpu.VMEM((C,), jnp.int32),
            pltpu.VMEM((C, D), jnp.float32),
            pltpu.VMEM((8, D), jnp.float32),
            pltpu.VMEM_SHARED((NP, D), jnp.float32),
            pltpu.SemaphoreType.DMA,
        ],
    )
    return f(h3, ridx)


def _sum2_body(p_ref, o_ref):
    o_ref[...] = p_ref[0] + p_ref[1]


def _sum_partials(parts):
    BN = 632
    grid = (NP // BN,)
    return pl.pallas_call(
        _sum2_body,
        grid=grid,
        in_specs=[pl.BlockSpec((2, BN, D), lambda i: (0, i, 0))],
        out_specs=pl.BlockSpec((BN, D), lambda i: (i, 0)),
        out_shape=jax.ShapeDtypeStruct((NP, D), jnp.float32),
    )(parts)


def kernel(x, edge_idx, W1, b1, W2, b2, W3, b3):
    ridx = edge_idx[0].astype(jnp.int32)
    cidxp = edge_idx[1].astype(jnp.int32) + N_NODES

    xab = _preproj(x, W1, b1).reshape(2 * N_NODES, D)
    hpre = _gather_add(xab, ridx, cidxp)
    h3 = _mlp_tail(hpre, W2, b2, W3, b3)
    parts = _segment_sum(h3, ridx)
    return _sum_partials(parts)[:N_NODES]
